# Initial kernel scaffold; baseline (speedup 1.0000x reference)
import jax
import jax.numpy as jnp
from jax import lax
from jax.experimental import pallas as pl
from jax.experimental.pallas import tpu as pltpu
from jax.experimental.pallas import tpu_sc as plsc

N_NODES = 10000
N_EDGES = 320000
D_FEAT = 128

NC = 2
NS = 16
L = 16

DH = D_FEAT // NC
EPT = N_EDGES // NS
K = 80
NCHUNK = EPT // K
RPT = N_NODES // NS


def _sc_body(xflat, x3, eidx, dtv, out3, sidx, sidx2, didx, rows, acc_sh,
             xe, ae, dtb, sem):
    c = lax.axis_index("c")
    s = lax.axis_index("s")
    row0 = s * RPT

    zv = jnp.zeros((L,), jnp.float32)

    @pl.loop(0, RPT)
    def _zero(r):
        for j in range(DH // L):
            ae[r, pl.ds(j * L, L)] = zv

    pltpu.sync_copy(ae, acc_sh.at[pl.ds(row0, RPT)])
    plsc.subcore_barrier()

    ebase = s * EPT

    @pl.loop(0, NCHUNK)
    def _edges(i):
        base = ebase + i * K
        pltpu.sync_copy(eidx.at[0, pl.ds(base, K)], sidx)
        pltpu.sync_copy(eidx.at[1, pl.ds(base, K)], didx)
        for j in range(K // L):
            sl = pl.ds(j * L, L)
            sidx2[sl] = sidx[sl] * 2 + c
        pltpu.async_copy(xflat.at[sidx2], rows, sem).wait()
        pltpu.sync_copy(rows, acc_sh.at[didx], add=True)

    plsc.subcore_barrier()

    pltpu.sync_copy(dtv, dtb)
    dt = dtb[...]
    pltpu.sync_copy(acc_sh.at[pl.ds(row0, RPT)], ae)
    pltpu.sync_copy(x3.at[pl.ds(row0, RPT), pl.ds(c, 1)], xe)

    @pl.loop(0, RPT)
    def _combine(r):
        for j in range(DH // L):
            sl = pl.ds(j * L, L)
            ae[r, sl] = xe[r, 0, sl] + dt * ae[r, sl]

    pltpu.sync_copy(ae, out3.at[pl.ds(row0, RPT), pl.ds(c, 1)])


def kernel(x, edge_index, start, end):
    xflat = x.reshape(NC * N_NODES, DH)
    x3 = x.reshape(N_NODES, NC, DH)
    dtv = jnp.full((L,), end - start, dtype=jnp.float32)

    out3 = pl.kernel(
        _sc_body,
        out_type=jax.ShapeDtypeStruct((N_NODES, NC, DH), jnp.float32),
        mesh=plsc.VectorSubcoreMesh(
            core_axis_name="c", subcore_axis_name="s",
            num_cores=NC, num_subcores=NS),
        scratch_types=[
            pltpu.VMEM((K,), jnp.int32),
            pltpu.VMEM((K,), jnp.int32),
            pltpu.VMEM((K,), jnp.int32),
            pltpu.VMEM((K, DH), jnp.float32),
            pltpu.VMEM_SHARED((N_NODES, DH), jnp.float32),
            pltpu.VMEM((RPT, 1, DH), jnp.float32),
            pltpu.VMEM((RPT, DH), jnp.float32),
            pltpu.VMEM((L,), jnp.float32),
            pltpu.SemaphoreType.DMA,
        ],
    )(xflat, x3, edge_index, dtv)

    return out3.reshape(N_NODES, D_FEAT)


# speedup vs baseline: 5.5719x; 5.5719x over previous
import jax
import jax.numpy as jnp
from jax import lax
from jax.experimental import pallas as pl
from jax.experimental.pallas import tpu as pltpu
from jax.experimental.pallas import tpu_sc as plsc

N_NODES = 10000
N_EDGES = 320000
D_FEAT = 128

NC = 2
NS = 16
L = 16

EPT = N_EDGES // (NC * NS)
K = 80
NCHUNK = EPT // K
RPT = N_NODES // NS
_CPR = 1000
_ZR = 125


def _sc_body(x, esrc, edst, part, sidx, didx, rows, acc_sh, zbuf, sem):
    c = lax.axis_index("c")
    s = lax.axis_index("s")
    row0 = s * RPT

    zv = jnp.zeros((L,), jnp.float32)

    @pl.loop(0, _ZR)
    def _zero(r):
        for j in range(D_FEAT // L):
            zbuf[r, pl.ds(j * L, L)] = zv

    @pl.loop(0, RPT // _ZR)
    def _zcopy(i):
        pltpu.sync_copy(zbuf, acc_sh.at[pl.ds(row0 + i * _ZR, _ZR)])

    plsc.subcore_barrier()

    ebase = (c * NS + s) * EPT

    @pl.loop(0, NCHUNK)
    def _edges(i):
        base = ebase + i * K
        pltpu.sync_copy(esrc.at[pl.ds(base, K)], sidx)
        pltpu.sync_copy(edst.at[pl.ds(base, K)], didx)
        pltpu.async_copy(x.at[sidx], rows, sem).wait()
        pltpu.sync_copy(rows, acc_sh.at[didx], add=True)

    plsc.subcore_barrier()

    @pl.when(s < N_NODES // _CPR)
    def _writeout():
        r0 = s * _CPR
        pltpu.sync_copy(acc_sh.at[pl.ds(r0, _CPR)], part.at[c, pl.ds(r0, _CPR)])


def _combine_body(dt_ref, x_ref, p_ref, o_ref):
    o_ref[...] = x_ref[...] + dt_ref[0] * (p_ref[0] + p_ref[1])


_BLK = 1000


def kernel(x, edge_index, start, end):
    part = pl.kernel(
        _sc_body,
        out_type=jax.ShapeDtypeStruct((NC, N_NODES, D_FEAT), jnp.float32),
        mesh=plsc.VectorSubcoreMesh(
            core_axis_name="c", subcore_axis_name="s",
            num_cores=NC, num_subcores=NS),
        scratch_types=[
            pltpu.VMEM((K,), jnp.int32),
            pltpu.VMEM((K,), jnp.int32),
            pltpu.VMEM((K, D_FEAT), jnp.float32),
            pltpu.VMEM_SHARED((N_NODES, D_FEAT), jnp.float32),
            pltpu.VMEM((_ZR, D_FEAT), jnp.float32),
            pltpu.SemaphoreType.DMA,
        ],
    )(x, edge_index[0], edge_index[1])

    dt = jnp.reshape(end - start, (1,)).astype(jnp.float32)
    out = pl.pallas_call(
        _combine_body,
        out_shape=jax.ShapeDtypeStruct((N_NODES, D_FEAT), jnp.float32),
        grid=(N_NODES // _BLK,),
        in_specs=[
            pl.BlockSpec(memory_space=pltpu.SMEM),
            pl.BlockSpec((_BLK, D_FEAT), lambda i: (i, 0)),
            pl.BlockSpec((NC, _BLK, D_FEAT), lambda i: (0, i, 0)),
        ],
        out_specs=pl.BlockSpec((_BLK, D_FEAT), lambda i: (i, 0)),
    )(dt, x, part)
    return out


# speedup vs baseline: 12.2740x; 2.2028x over previous
import jax
import jax.numpy as jnp
from jax import lax
from jax.experimental import pallas as pl
from jax.experimental.pallas import tpu as pltpu
from jax.experimental.pallas import tpu_sc as plsc

N_NODES = 10000
N_EDGES = 320000
D_FEAT = 128

NC = 2
NS = 16
L = 16

EPT = N_EDGES // (NC * NS)
K = 80
NCHUNK = EPT // K
_CPR = 1000
RPT = N_NODES // NS


def _sc_body(x, esrc, edst, part, sbuf, dbuf, rows0, rows1, acc_sh,
             semi, sem0, sem1):
    c = lax.axis_index("c")
    s = lax.axis_index("s")
    wid = c * NS + s

    da = pltpu.async_copy(esrc.at[wid], sbuf, semi)
    db = pltpu.async_copy(edst.at[wid], dbuf, semi)

    zv = jnp.zeros((L,), jnp.float32)

    @pl.loop(0, K)
    def _zero(r):
        for j in range(D_FEAT // L):
            rows0[r, pl.ds(j * L, L)] = zv

    row0 = s * RPT

    @pl.loop(0, RPT // K)
    def _zcopy(i):
        pltpu.sync_copy(rows0, acc_sh.at[pl.ds(row0 + i * K, K)])

    pltpu.sync_copy(rows0.at[pl.ds(0, RPT - (RPT // K) * K)],
                    acc_sh.at[pl.ds(row0 + (RPT // K) * K,
                                    RPT - (RPT // K) * K)])

    da.wait()
    db.wait()
    plsc.subcore_barrier()

    pltpu.async_copy(x.at[sbuf.at[pl.ds(0, K)]], rows0, sem0)

    @pl.loop(0, NCHUNK - 1, step=2)
    def _edges(i):
        d1 = pltpu.async_copy(x.at[sbuf.at[pl.ds((i + 1) * K, K)]],
                              rows1, sem1)
        pltpu.make_async_copy(x.at[pl.ds(0, K)], rows0, sem0).wait()
        pltpu.sync_copy(rows0, acc_sh.at[dbuf.at[i]], add=True)
        pltpu.async_copy(x.at[sbuf.at[pl.ds((i + 2) * K, K)]], rows0, sem0)
        d1.wait()
        pltpu.sync_copy(rows1, acc_sh.at[dbuf.at[i + 1]], add=True)

    pltpu.make_async_copy(x.at[pl.ds(0, K)], rows0, sem0).wait()
    pltpu.sync_copy(rows0, acc_sh.at[dbuf.at[NCHUNK - 1]], add=True)

    plsc.subcore_barrier()

    @pl.when(s < N_NODES // _CPR)
    def _writeout():
        r0 = s * _CPR
        pltpu.sync_copy(acc_sh.at[pl.ds(r0, _CPR)],
                        part.at[c, pl.ds(r0, _CPR)])


def _combine_body(dt_ref, x_ref, p_ref, o_ref):
    o_ref[...] = x_ref[...] + dt_ref[0] * (p_ref[0] + p_ref[1])


_BLK = 1000


def kernel(x, edge_index, start, end):
    esrc = edge_index[0].reshape(NC * NS, EPT)
    edst = edge_index[1].reshape(NC * NS, NCHUNK, K)

    part = pl.kernel(
        _sc_body,
        out_type=jax.ShapeDtypeStruct((NC, N_NODES, D_FEAT), jnp.float32),
        mesh=plsc.VectorSubcoreMesh(
            core_axis_name="c", subcore_axis_name="s",
            num_cores=NC, num_subcores=NS),
        scratch_types=[
            pltpu.VMEM((EPT,), jnp.int32),
            pltpu.VMEM((NCHUNK, K), jnp.int32),
            pltpu.VMEM((K, D_FEAT), jnp.float32),
            pltpu.VMEM((K, D_FEAT), jnp.float32),
            pltpu.VMEM_SHARED((N_NODES, D_FEAT), jnp.float32),
            pltpu.SemaphoreType.DMA,
            pltpu.SemaphoreType.DMA,
            pltpu.SemaphoreType.DMA,
        ],
    )(x, esrc, edst)

    dt = jnp.reshape(end - start, (1,)).astype(jnp.float32)
    out = pl.pallas_call(
        _combine_body,
        out_shape=jax.ShapeDtypeStruct((N_NODES, D_FEAT), jnp.float32),
        grid=(N_NODES // _BLK,),
        in_specs=[
            pl.BlockSpec(memory_space=pltpu.SMEM),
            pl.BlockSpec((_BLK, D_FEAT), lambda i: (i, 0)),
            pl.BlockSpec((NC, _BLK, D_FEAT), lambda i: (0, i, 0)),
        ],
        out_specs=pl.BlockSpec((_BLK, D_FEAT), lambda i: (i, 0)),
    )(dt, x, part)
    return out
